# Initial kernel scaffold; baseline (speedup 1.0000x reference)
import jax
import jax.numpy as jnp
from jax.experimental import pallas as pl


def kernel(inputs, codebook):
    raise NotImplementedError("write your pallas kernel here")


# speedup vs baseline: 3.7135x; 3.7135x over previous
import functools

import jax
import jax.numpy as jnp
from jax.experimental import pallas as pl

_B = 2048
_CODE_DIM = 32
_K = 512
_TEMP = 100000000.0
_ROWS = _B * _CODE_DIM
_BLK = 1024


def _vq_body(x_ref, cb_ref, soft_ref, hard_ref, q_ref):
    x = x_ref[:, :]
    cb = cb_ref[0, :]
    diff = x - cb[None, :]
    d = jnp.exp(-jnp.sqrt(diff * diff))
    t = _TEMP * d
    m = jnp.max(t, axis=1, keepdims=True)
    e = jnp.exp(t - m)
    s = jnp.sum(e, axis=1, keepdims=True)
    soft = e / s
    soft_ref[:, :] = soft
    dmax = jnp.max(d, axis=1, keepdims=True)
    iota = jax.lax.broadcasted_iota(jnp.int32, (_BLK, _K), 1)
    idx = jnp.min(jnp.where(d == dmax, iota, _K), axis=1, keepdims=True)
    hard_ref[:, :] = (iota == idx).astype(jnp.float32)
    q_ref[:, :] = jnp.sum(soft * cb[None, :], axis=1, keepdims=True)


@functools.partial(jax.jit)
def kernel(inputs, codebook):
    x = inputs.reshape(_ROWS, 1)
    grid = (_ROWS // _BLK,)
    soft, hard, q = pl.pallas_call(
        _vq_body,
        grid=grid,
        in_specs=[
            pl.BlockSpec((_BLK, 1), lambda i: (i, 0)),
            pl.BlockSpec((1, _K), lambda i: (0, 0)),
        ],
        out_specs=[
            pl.BlockSpec((_BLK, _K), lambda i: (i, 0)),
            pl.BlockSpec((_BLK, _K), lambda i: (i, 0)),
            pl.BlockSpec((_BLK, 1), lambda i: (i, 0)),
        ],
        out_shape=[
            jax.ShapeDtypeStruct((_ROWS, _K), jnp.float32),
            jax.ShapeDtypeStruct((_ROWS, _K), jnp.float32),
            jax.ShapeDtypeStruct((_ROWS, 1), jnp.float32),
        ],
    )(x, codebook)
    soft = soft.reshape(_B, _CODE_DIM, _K)
    hard = hard.reshape(_B, _CODE_DIM, _K)
    q = q.reshape(_B, _CODE_DIM)
    return (soft, hard, q)


# speedup vs baseline: 4.1397x; 1.1148x over previous
import functools

import jax
import jax.numpy as jnp
from jax.experimental import pallas as pl

_B = 2048
_CODE_DIM = 32
_K = 512
_TEMP = 100000000.0
_ROWS = _B * _CODE_DIM
_BLK = 1024


def _vq_body(x_ref, cb_ref, soft_ref, hard_ref, q_ref):
    x = x_ref[:, :]
    cb = cb_ref[0, :]
    d = jnp.exp(-jnp.abs(x - cb[None, :]))
    dmax = jnp.max(d, axis=1, keepdims=True)
    m = _TEMP * dmax
    t = _TEMP * d
    e = jnp.exp(t - m)
    s = jnp.sum(e, axis=1, keepdims=True)
    soft = e / s
    soft_ref[:, :] = soft
    iota = jax.lax.broadcasted_iota(jnp.int32, (_BLK, _K), 1)
    idx = jnp.min(jnp.where(d == dmax, iota, _K), axis=1, keepdims=True)
    hard_ref[:, :] = (iota == idx).astype(jnp.float32)
    q_ref[:, :] = jax.lax.dot_general(
        soft, cb[:, None], (((1,), (0,)), ((), ())),
        preferred_element_type=jnp.float32)


@functools.partial(jax.jit)
def kernel(inputs, codebook):
    x = inputs.reshape(_ROWS, 1)
    grid = (_ROWS // _BLK,)
    soft, hard, q = pl.pallas_call(
        _vq_body,
        grid=grid,
        in_specs=[
            pl.BlockSpec((_BLK, 1), lambda i: (i, 0)),
            pl.BlockSpec((1, _K), lambda i: (0, 0)),
        ],
        out_specs=[
            pl.BlockSpec((_BLK, _K), lambda i: (i, 0)),
            pl.BlockSpec((_BLK, _K), lambda i: (i, 0)),
            pl.BlockSpec((_BLK, 1), lambda i: (i, 0)),
        ],
        out_shape=[
            jax.ShapeDtypeStruct((_ROWS, _K), jnp.float32),
            jax.ShapeDtypeStruct((_ROWS, _K), jnp.float32),
            jax.ShapeDtypeStruct((_ROWS, 1), jnp.float32),
        ],
    )(x, codebook)
    soft = soft.reshape(_B, _CODE_DIM, _K)
    hard = hard.reshape(_B, _CODE_DIM, _K)
    q = q.reshape(_B, _CODE_DIM)
    return (soft, hard, q)


# speedup vs baseline: 4.5445x; 1.0978x over previous
import functools

import jax
import jax.numpy as jnp
from jax.experimental import pallas as pl

_B = 2048
_CODE_DIM = 32
_K = 512
_TEMP = 100000000.0
_ROWS = _B * _CODE_DIM
_BLK = 2048


def _vq_body(x_ref, cb_ref, soft_ref, hard_ref, q_ref):
    x = x_ref[:, :]
    cb = cb_ref[0, :]
    d = jnp.exp(-jnp.abs(x - cb[None, :]))
    dmax = jnp.max(d, axis=1, keepdims=True)
    m = _TEMP * dmax
    t = _TEMP * d
    e = jnp.exp(t - m)
    s = jnp.sum(e, axis=1, keepdims=True)
    soft = e / s
    soft_ref[:, :] = soft
    iota = jax.lax.broadcasted_iota(jnp.int32, (_BLK, _K), 1)
    idx = jnp.min(jnp.where(d == dmax, iota, _K), axis=1, keepdims=True)
    hard_ref[:, :] = (iota == idx).astype(jnp.float32)
    q_ref[:, :] = jax.lax.dot_general(
        soft, cb[:, None], (((1,), (0,)), ((), ())),
        preferred_element_type=jnp.float32)


@functools.partial(jax.jit)
def kernel(inputs, codebook):
    x = inputs.reshape(_ROWS, 1)
    grid = (_ROWS // _BLK,)
    soft, hard, q = pl.pallas_call(
        _vq_body,
        grid=grid,
        in_specs=[
            pl.BlockSpec((_BLK, 1), lambda i: (i, 0)),
            pl.BlockSpec((1, _K), lambda i: (0, 0)),
        ],
        out_specs=[
            pl.BlockSpec((_BLK, _K), lambda i: (i, 0)),
            pl.BlockSpec((_BLK, _K), lambda i: (i, 0)),
            pl.BlockSpec((_BLK, 1), lambda i: (i, 0)),
        ],
        out_shape=[
            jax.ShapeDtypeStruct((_ROWS, _K), jnp.float32),
            jax.ShapeDtypeStruct((_ROWS, _K), jnp.float32),
            jax.ShapeDtypeStruct((_ROWS, 1), jnp.float32),
        ],
    )(x, codebook)
    soft = soft.reshape(_B, _CODE_DIM, _K)
    hard = hard.reshape(_B, _CODE_DIM, _K)
    q = q.reshape(_B, _CODE_DIM)
    return (soft, hard, q)


# speedup vs baseline: 4.7856x; 1.0531x over previous
import functools

import jax
import jax.numpy as jnp
from jax.experimental import pallas as pl

_B = 2048
_CODE_DIM = 32
_K = 512
_TEMP = 100000000.0
_ROWS = _B * _CODE_DIM
_BLK = 2048


def _vq_body(x_ref, cb_ref, soft_ref, hard_ref, q_ref):
    x = x_ref[:, :]
    cb = cb_ref[0, :]
    d = jnp.exp(-jnp.abs(x - cb[None, :]))
    dmax = jnp.max(d, axis=1, keepdims=True)
    m = _TEMP * dmax
    t = _TEMP * d
    e = jnp.exp(t - m)
    s = jnp.sum(e, axis=1, keepdims=True)
    soft = e / s
    soft_ref[:, :] = soft
    iota = jax.lax.broadcasted_iota(jnp.int32, (_BLK, _K), 1)
    idx = jnp.min(jnp.where(d == dmax, iota, _K), axis=1, keepdims=True)
    hard_ref[:, :] = (iota == idx).astype(jnp.float32)
    q = jax.lax.dot_general(
        soft, cb[:, None], (((1,), (0,)), ((), ())),
        preferred_element_type=jnp.float32)
    q_ref[:, :] = q.reshape(_BLK // 128, 128)


@functools.partial(jax.jit)
def kernel(inputs, codebook):
    x = inputs.reshape(_ROWS, 1)
    grid = (_ROWS // _BLK,)
    soft, hard, q = pl.pallas_call(
        _vq_body,
        grid=grid,
        in_specs=[
            pl.BlockSpec((_BLK, 1), lambda i: (i, 0)),
            pl.BlockSpec((1, _K), lambda i: (0, 0)),
        ],
        out_specs=[
            pl.BlockSpec((_BLK, _K), lambda i: (i, 0)),
            pl.BlockSpec((_BLK, _K), lambda i: (i, 0)),
            pl.BlockSpec((_BLK // 128, 128), lambda i: (i, 0)),
        ],
        out_shape=[
            jax.ShapeDtypeStruct((_ROWS, _K), jnp.float32),
            jax.ShapeDtypeStruct((_ROWS, _K), jnp.float32),
            jax.ShapeDtypeStruct((_ROWS // 128, 128), jnp.float32),
        ],
    )(x, codebook)
    soft = soft.reshape(_B, _CODE_DIM, _K)
    hard = hard.reshape(_B, _CODE_DIM, _K)
    q = q.reshape(_B, _CODE_DIM)
    return (soft, hard, q)
